# baseline (speedup 1.0000x reference)
import functools

import jax
import jax.numpy as jnp
from jax import lax
from jax.experimental import pallas as pl
from jax.experimental.pallas import tpu as pltpu
from jax.experimental.pallas import tpu_sc as plsc

_NC = 2
_NS = 16
_NW = _NC * _NS
_LANES = 16


@functools.partial(jax.jit, static_argnames=("n_chunks", "chunk", "t", "d"))
def _sc_embed(ids_3d, token_table, pos_table, *, n_chunks, chunk, t, d):
    n_rows = _NW * n_chunks * chunk
    rows_per_w = n_chunks * chunk
    mesh = plsc.VectorSubcoreMesh(core_axis_name="c", subcore_axis_name="s")

    @functools.partial(
        pl.kernel,
        out_type=jax.ShapeDtypeStruct((n_rows, d), jnp.float32),
        mesh=mesh,
        scratch_types=[
            pltpu.VMEM((chunk,), jnp.int32),
            pltpu.VMEM((chunk, d), jnp.float32),
            pltpu.VMEM((t, d), jnp.float32),
            pltpu.SemaphoreType.DMA,
        ],
    )
    def body(ids_hbm, table_hbm, pos_hbm, out_hbm, idx_v, rows_v, pos_v, sem):
        wid = lax.axis_index("s") * _NC + lax.axis_index("c")
        base = wid * rows_per_w
        pltpu.sync_copy(pos_hbm, pos_v)

        n_groups = chunk // t
        slabs = d // _LANES

        def do_chunk(c, _):
            rowbase = base + c * chunk
            pltpu.sync_copy(ids_hbm.at[wid, c], idx_v)
            pltpu.async_copy(table_hbm.at[idx_v], rows_v, sem).wait()

            def add_group(g, _):
                r0 = g * t
                for k in range(t):
                    for j in range(slabs):
                        sl = pl.ds(j * _LANES, _LANES)
                        rows_v[r0 + k, sl] = rows_v[r0 + k, sl] + pos_v[k, sl]
                return 0

            lax.fori_loop(0, n_groups, add_group, 0)
            pltpu.sync_copy(rows_v, out_hbm.at[pl.ds(rowbase, chunk)])
            return 0

        lax.fori_loop(0, n_chunks, do_chunk, 0)

    return body(ids_3d, token_table, pos_table)


def kernel(input_ids, token_table, pos_table):
    bq, tq = input_ids.shape
    vocab, d = token_table.shape
    n = bq * tq
    chunk = 128
    assert n % (_NW * chunk) == 0 and chunk % tq == 0 and d % _LANES == 0
    n_chunks = n // (_NW * chunk)
    ids_3d = input_ids.astype(jnp.int32).reshape(_NW, n_chunks, chunk)
    out = _sc_embed(ids_3d, token_table, pos_table,
                    n_chunks=n_chunks, chunk=chunk, t=tq, d=d)
    return out.reshape(bq, tq, d)


# speedup vs baseline: 3.1210x; 3.1210x over previous
import functools

import jax
import jax.numpy as jnp
from jax import lax
from jax.experimental import pallas as pl
from jax.experimental.pallas import tpu as pltpu
from jax.experimental.pallas import tpu_sc as plsc

_NC = 2
_NS = 16
_NW = _NC * _NS
_LANES = 16
_NBUF = 4


@functools.partial(jax.jit, static_argnames=("n_chunks", "chunk", "t", "d"))
def _sc_embed(ids_3d, token_table, pos_table, *, n_chunks, chunk, t, d):
    n_rows = _NW * n_chunks * chunk
    rows_per_w = n_chunks * chunk
    n_rounds = n_chunks // _NBUF
    n_groups = chunk // t
    slabs = d // _LANES
    mesh = plsc.VectorSubcoreMesh(core_axis_name="c", subcore_axis_name="s")

    @functools.partial(
        pl.kernel,
        out_type=jax.ShapeDtypeStruct((n_rows, d), jnp.float32),
        mesh=mesh,
        scratch_types=[
            pltpu.VMEM((n_chunks, chunk), jnp.int32),
            pltpu.VMEM((_NBUF, chunk, d), jnp.float32),
            pltpu.VMEM((t, d), jnp.float32),
        ]
        + [pltpu.SemaphoreType.DMA] * (2 * _NBUF),
    )
    def body(ids_hbm, table_hbm, pos_hbm, out_hbm, idx_v, rows, pos_v, *sems):
        sgs, sos = sems[:_NBUF], sems[_NBUF:]
        wid = lax.axis_index("s") * _NC + lax.axis_index("c")
        base = wid * rows_per_w
        pltpu.sync_copy(pos_hbm, pos_v)
        pltpu.sync_copy(ids_hbm.at[wid], idx_v)

        def ga(c, b):
            return pltpu.make_async_copy(
                table_hbm.at[idx_v.at[c]], rows.at[b], sgs[b])

        def wb(c, b):
            return pltpu.make_async_copy(
                rows.at[b], out_hbm.at[pl.ds(base + c * chunk, chunk)], sos[b])

        def add_chunk(b):
            for j in range(slabs):
                sl = pl.ds(j * _LANES, _LANES)
                pvs = tuple(pos_v[k, sl] for k in range(t))

                def grp(g, pvs):
                    r0 = g * t
                    for k in range(t):
                        rows[b, r0 + k, sl] = rows[b, r0 + k, sl] + pvs[k]
                    return pvs

                lax.fori_loop(0, n_groups, grp, pvs)

        def process(c, b):
            ga(c, b).wait()
            add_chunk(b)
            wb(c, b).start()

        ga(0, 0).start()
        ga(1, 1).start()

        for k in range(_NBUF):
            process(k, k)
            if k < 2:
                ga(k + 2, k + 2).start()
            else:
                wb(k - 2, k - 2).wait()
                ga(k + 2, k - 2).start()

        def mid(p, _):
            c0 = p * _NBUF
            for k in range(_NBUF):
                c = c0 + k
                process(c, k)
                b2 = (k + 2) % _NBUF
                wb(c - 2, b2).wait()
                ga(c + 2, b2).start()
            return 0

        lax.fori_loop(1, n_rounds - 1, mid, 0)

        cL = (n_rounds - 1) * _NBUF
        for k in range(_NBUF):
            c = cL + k
            process(c, k)
            if k < 2:
                b2 = (k + 2) % _NBUF
                wb(c - 2, b2).wait()
                ga(c + 2, b2).start()

        for k in range(_NBUF):
            wb(cL + k, k).wait()

    return body(ids_3d, token_table, pos_table)


def kernel(input_ids, token_table, pos_table):
    bq, tq = input_ids.shape
    vocab, d = token_table.shape
    n = bq * tq
    chunk = 128
    assert n % (_NW * chunk) == 0 and chunk % tq == 0 and d % _LANES == 0
    n_chunks = n // (_NW * chunk)
    assert n_chunks % _NBUF == 0 and n_chunks // _NBUF >= 2
    ids_3d = input_ids.astype(jnp.int32).reshape(_NW, n_chunks, chunk)
    out = _sc_embed(ids_3d, token_table, pos_table,
                    n_chunks=n_chunks, chunk=chunk, t=tq, d=d)
    return out.reshape(bq, tq, d)


# speedup vs baseline: 3.2043x; 1.0267x over previous
import functools

import jax
import jax.numpy as jnp
from jax import lax
from jax.experimental import pallas as pl
from jax.experimental.pallas import tpu as pltpu
from jax.experimental.pallas import tpu_sc as plsc

_NC = 2
_NS = 16
_NW = _NC * _NS
_LANES = 16
_NBUF = 4


@functools.partial(jax.jit, static_argnames=("n_chunks", "chunk", "t", "d"))
def _sc_embed(ids_3d, token_table, pos_table, *, n_chunks, chunk, t, d):
    n_rows = _NW * n_chunks * chunk
    rows_per_w = n_chunks * chunk
    n_rounds = n_chunks // _NBUF
    n_groups = chunk // t
    slabs = d // _LANES
    mesh = plsc.VectorSubcoreMesh(core_axis_name="c", subcore_axis_name="s")

    @functools.partial(
        pl.kernel,
        out_type=jax.ShapeDtypeStruct((n_rows, d), jnp.float32),
        mesh=mesh,
        scratch_types=[
            pltpu.VMEM((n_chunks, chunk), jnp.int32),
            pltpu.VMEM((_NBUF, chunk, d), jnp.float32),
            pltpu.VMEM((t, d), jnp.float32),
        ]
        + [pltpu.SemaphoreType.DMA] * (2 * _NBUF),
    )
    def body(ids_hbm, table_hbm, pos_hbm, out_hbm, idx_v, rows, pos_v, *sems):
        sgs, sos = sems[:_NBUF], sems[_NBUF:]
        wid = lax.axis_index("s") * _NC + lax.axis_index("c")
        base = wid * rows_per_w
        pltpu.sync_copy(pos_hbm, pos_v)
        pltpu.sync_copy(ids_hbm.at[wid], idx_v)

        def ga(c, b):
            return pltpu.make_async_copy(
                table_hbm.at[idx_v.at[c]], rows.at[b], sgs[b])

        def wb(c, b):
            return pltpu.make_async_copy(
                rows.at[b], out_hbm.at[pl.ds(base + c * chunk, chunk)], sos[b])

        def add_chunk(b):
            for j in range(slabs):
                sl = pl.ds(j * _LANES, _LANES)
                pvs = tuple(pos_v[k, sl] for k in range(t))

                def grp(g, pvs):
                    r0 = g * t
                    for k in range(t):
                        rows[b, r0 + k, sl] = rows[b, r0 + k, sl] + pvs[k]
                    return pvs

                lax.fori_loop(0, n_groups, grp, pvs)

        ga(0, 0).start()
        ga(1, 1).start()

        for k in range(_NBUF):
            ga(k, k).wait()
            if k < 2:
                ga(k + 2, k + 2).start()
            else:
                wb(k - 2, k - 2).wait()
                ga(k + 2, k - 2).start()
            add_chunk(k)
            wb(k, k).start()

        def mid(p, _):
            c0 = p * _NBUF
            for k in range(_NBUF):
                c = c0 + k
                ga(c, k).wait()
                b2 = (k + 2) % _NBUF
                wb(c - 2, b2).wait()
                ga(c + 2, b2).start()
                add_chunk(k)
                wb(c, k).start()
            return 0

        lax.fori_loop(1, n_rounds - 1, mid, 0)

        cL = (n_rounds - 1) * _NBUF
        for k in range(_NBUF):
            c = cL + k
            ga(c, k).wait()
            if k < 2:
                b2 = (k + 2) % _NBUF
                wb(c - 2, b2).wait()
                ga(c + 2, b2).start()
            add_chunk(k)
            wb(c, k).start()

        for k in range(_NBUF):
            wb(cL + k, k).wait()

    return body(ids_3d, token_table, pos_table)


def kernel(input_ids, token_table, pos_table):
    bq, tq = input_ids.shape
    vocab, d = token_table.shape
    n = bq * tq
    chunk = 128
    assert n % (_NW * chunk) == 0 and chunk % tq == 0 and d % _LANES == 0
    n_chunks = n // (_NW * chunk)
    assert n_chunks % _NBUF == 0 and n_chunks // _NBUF >= 2
    ids_3d = input_ids.astype(jnp.int32).reshape(_NW, n_chunks, chunk)
    out = _sc_embed(ids_3d, token_table, pos_table,
                    n_chunks=n_chunks, chunk=chunk, t=tq, d=d)
    return out.reshape(bq, tq, d)
